# Initial kernel scaffold; baseline (speedup 1.0000x reference)
import jax
import jax.numpy as jnp
from jax.experimental import pallas as pl


def kernel(p0, p1, p2, W, bconv, centroids):
    raise NotImplementedError("write your pallas kernel here")


# speedup vs baseline: 16.0361x; 16.0361x over previous
import functools

import jax
import jax.numpy as jnp
from jax.experimental import pallas as pl


def _avg_pool3(x):
    s = jax.lax.reduce_window(x, 0.0, jax.lax.add, (1, 1, 3, 3), (1, 1, 1, 1),
                              ((0, 0), (0, 0), (1, 1), (1, 1)))
    return s / 9.0


def _fused_body(inp_ref, wt_ref, ct_ref, cent_ref, out_ref):
    x = inp_ref[0]
    e = jnp.dot(x, wt_ref[...], preferred_element_type=jnp.float32) + ct_ref[...]
    feats = jnp.sum(e * e, axis=1, keepdims=True)
    cent = cent_ref[...]
    centers = jnp.sum(cent * cent, axis=0, keepdims=True)
    d2 = feats + centers - 2.0 * jnp.dot(e, cent, preferred_element_type=jnp.float32)

    iota = jax.lax.broadcasted_iota(jnp.int32, d2.shape, 1)
    cur = d2
    mins = []
    for _ in range(3):
        mins.append(jnp.min(cur, axis=1))
        am = jnp.argmin(cur, axis=1)
        cur = jnp.where(iota == am[:, None], jnp.inf, cur)
    d0 = jnp.sqrt(jnp.maximum(mins[0], 1e-12))
    d1 = jnp.sqrt(jnp.maximum(mins[1], 1e-12))
    d2s = jnp.sqrt(jnp.maximum(mins[2], 1e-12))
    sm0 = 1.0 / (1.0 + jnp.exp(d0 - d1) + jnp.exp(d0 - d2s))
    out_ref[0, 0] = sm0 * d0


@functools.partial(jax.jit, static_argnums=())
def kernel(p0, p1, p2, W, bconv, centroids):
    b = p0.shape[0]
    h, w = p0.shape[2], p0.shape[3]
    hw = h * w
    c = centroids.shape[0]
    n = centroids.shape[1]

    a0 = _avg_pool3(p0)
    a1 = _avg_pool3(p1)
    a2 = _avg_pool3(p2)
    a1 = jax.image.resize(a1, (b, a1.shape[1], h, w), method='bilinear')
    a2 = jax.image.resize(a2, (b, a2.shape[1], h, w), method='bilinear')
    feat = jnp.concatenate([a0, a1, a2], axis=1)
    inp = feat.reshape(b, c, hw).transpose(0, 2, 1)

    xx = (jnp.arange(w, dtype=jnp.float32) / (w - 1)) * 2.0 - 1.0
    yy = (jnp.arange(h, dtype=jnp.float32) / (h - 1)) * 2.0 - 1.0
    grid_x = jnp.tile(xx, h)
    grid_y = jnp.repeat(yy, w)
    ct = (grid_x[:, None] * W[None, :, c] + grid_y[:, None] * W[None, :, c + 1]
          + bconv[None, :])
    wt = W[:, :c].T

    bm = 448 if hw % 448 == 0 else hw
    nblk = hw // bm
    score = pl.pallas_call(
        _fused_body,
        grid=(b, nblk),
        in_specs=[
            pl.BlockSpec((1, bm, c), lambda i, j: (i, j, 0)),
            pl.BlockSpec((c, c), lambda i, j: (0, 0)),
            pl.BlockSpec((bm, c), lambda i, j: (j, 0)),
            pl.BlockSpec((c, n), lambda i, j: (0, 0)),
        ],
        out_specs=pl.BlockSpec((1, 1, bm), lambda i, j: (i * nblk + j, 0, 0)),
        out_shape=jax.ShapeDtypeStruct((b * nblk, 1, bm), jnp.float32),
    )(inp, wt, ct, centroids)

    return score.reshape(b, 1, h, w)


# speedup vs baseline: 18.1327x; 1.1307x over previous
import functools

import jax
import jax.numpy as jnp
from jax.experimental import pallas as pl


def _avg_pool3(x):
    s = jax.lax.reduce_window(x, 0.0, jax.lax.add, (1, 1, 3, 3), (1, 1, 1, 1),
                              ((0, 0), (0, 0), (1, 1), (1, 1)))
    return s / 9.0


def _fused_body(inp_ref, wt_ref, ct_ref, cent_ref, centers_ref, out_ref):
    x = inp_ref[0]
    e = jnp.dot(x, wt_ref[...], preferred_element_type=jnp.float32) + ct_ref[...]
    feats = jnp.sum(e * e, axis=1, keepdims=True)
    eb = e.astype(jnp.bfloat16)
    d2 = (feats + centers_ref[...]
          - 2.0 * jnp.dot(eb, cent_ref[...], preferred_element_type=jnp.float32))

    iota = jax.lax.broadcasted_iota(jnp.int32, d2.shape, 1)
    cur = d2
    mins = []
    for _ in range(3):
        mins.append(jnp.min(cur, axis=1))
        am = jnp.argmin(cur, axis=1)
        cur = jnp.where(iota == am[:, None], jnp.inf, cur)
    d0 = jnp.sqrt(jnp.maximum(mins[0], 1e-12))
    d1 = jnp.sqrt(jnp.maximum(mins[1], 1e-12))
    d2s = jnp.sqrt(jnp.maximum(mins[2], 1e-12))
    sm0 = 1.0 / (1.0 + jnp.exp(d0 - d1) + jnp.exp(d0 - d2s))
    out_ref[0, 0] = sm0 * d0


@functools.partial(jax.jit, static_argnums=())
def kernel(p0, p1, p2, W, bconv, centroids):
    b = p0.shape[0]
    h, w = p0.shape[2], p0.shape[3]
    hw = h * w
    c = centroids.shape[0]
    n = centroids.shape[1]

    a0 = _avg_pool3(p0)
    a1 = _avg_pool3(p1)
    a2 = _avg_pool3(p2)
    a1 = jax.image.resize(a1, (b, a1.shape[1], h, w), method='bilinear')
    a2 = jax.image.resize(a2, (b, a2.shape[1], h, w), method='bilinear')
    feat = jnp.concatenate([a0, a1, a2], axis=1)
    inp = feat.reshape(b, c, hw).transpose(0, 2, 1).astype(jnp.bfloat16)

    xx = (jnp.arange(w, dtype=jnp.float32) / (w - 1)) * 2.0 - 1.0
    yy = (jnp.arange(h, dtype=jnp.float32) / (h - 1)) * 2.0 - 1.0
    grid_x = jnp.tile(xx, h)
    grid_y = jnp.repeat(yy, w)
    ct = (grid_x[:, None] * W[None, :, c] + grid_y[:, None] * W[None, :, c + 1]
          + bconv[None, :])
    wt = W[:, :c].T.astype(jnp.bfloat16)
    centb = centroids.astype(jnp.bfloat16)
    centers = jnp.sum(centroids * centroids, axis=0, keepdims=True)

    bm = 448 if hw % 448 == 0 else hw
    nblk = hw // bm
    score = pl.pallas_call(
        _fused_body,
        grid=(b, nblk),
        in_specs=[
            pl.BlockSpec((1, bm, c), lambda i, j: (i, j, 0)),
            pl.BlockSpec((c, c), lambda i, j: (0, 0)),
            pl.BlockSpec((bm, c), lambda i, j: (j, 0)),
            pl.BlockSpec((c, n), lambda i, j: (0, 0)),
            pl.BlockSpec((1, n), lambda i, j: (0, 0)),
        ],
        out_specs=pl.BlockSpec((1, 1, bm), lambda i, j: (i * nblk + j, 0, 0)),
        out_shape=jax.ShapeDtypeStruct((b * nblk, 1, bm), jnp.float32),
    )(inp, wt, ct, centb, centers)

    return score.reshape(b, 1, h, w)
